# baseline (speedup 1.0000x reference)
import functools

import jax
import jax.numpy as jnp
from jax import lax
from jax.experimental import pallas as pl
from jax.experimental.pallas import tpu as pltpu
from jax.experimental.pallas import tpu_sc as plsc

N = 10000
C = 128
K = 32
NC = 2
NS = 16
NW = NC * NS
BATCH = 4
RPG = BATCH * K
CB = C // 16
KU = 8

B = 80
OUTS = 1
STEP = 2 * OUTS
NB_TOT = N // BATCH
IDX_ROWS = NB_TOT + 8

XSTRIPE = 632

assert B % STEP == 0

_mesh = plsc.VectorSubcoreMesh(core_axis_name="c", subcore_axis_name="s")


@functools.partial(
    pl.kernel,
    mesh=_mesh,
    out_type=jax.ShapeDtypeStruct((N, C), jnp.float32),
    scratch_types=[
        pltpu.VMEM_SHARED((N, C), jnp.float32),
        pltpu.VMEM((48, RPG), jnp.int32),
        pltpu.VMEM((2, RPG, C), jnp.float32),
        pltpu.VMEM((2, OUTS * BATCH, C), jnp.float32),
        pltpu.VMEM((K, C), jnp.float32),
        pltpu.SemaphoreType.DMA,
        pltpu.SemaphoreType.DMA,
        pltpu.SemaphoreType.DMA,
        pltpu.SemaphoreType.DMA,
    ],
)
def _cluster_conv(xt_hbm, idx_hbm, wt_hbm, out_hbm,
                  xt_s, idx_v, rows_v, out_v, wt_v,
                  semr0, semr1, semf0, semf1):
    cid = lax.axis_index("c")
    sid = lax.axis_index("s")
    wid = sid * NC + cid
    base_b = jnp.minimum(wid * B, NB_TOT - B)
    abase = jnp.minimum(wid * B, (NB_TOT - B) // 8 * 8)
    goff = base_b - abase

    @pl.when(sid < NS - 1)
    def _():
        pltpu.sync_copy(xt_hbm.at[pl.ds(sid * XSTRIPE, XSTRIPE), :],
                        xt_s.at[pl.ds(sid * XSTRIPE, XSTRIPE), :])

    @pl.when(sid == NS - 1)
    def _():
        pltpu.sync_copy(
            xt_hbm.at[pl.ds((NS - 1) * XSTRIPE, N - (NS - 1) * XSTRIPE), :],
            xt_s.at[pl.ds((NS - 1) * XSTRIPE, N - (NS - 1) * XSTRIPE), :])

    pltpu.sync_copy(idx_hbm.at[pl.ds(abase, 48), :], idx_v)
    pltpu.sync_copy(wt_hbm, wt_v)
    plsc.subcore_barrier()

    semr = (semr0, semr1)
    semf = (semf0, semf1)

    def fire_rows(g, buf):
        rel = goff + g
        slot = jnp.where(rel >= 48, rel - 48, rel)
        pltpu.async_copy(xt_s.at[idx_v.at[slot]], rows_v.at[buf],
                         semr[buf])

    def wait_rows(buf):
        pltpu.make_async_copy(
            xt_s.at[idx_v.at[0]], rows_v.at[buf], semr[buf]).wait()

    def fire_flush(slot, g_first):
        pltpu.async_copy(
            out_v.at[slot],
            out_hbm.at[pl.ds((base_b + g_first) * BATCH, OUTS * BATCH), :],
            semf[slot])

    def wait_flush(slot):
        pltpu.make_async_copy(
            out_v.at[slot],
            out_hbm.at[pl.ds(base_b * BATCH, OUTS * BATCH), :],
            semf[slot]).wait()

    def compute(g, buf, slot, off):
        HB = CB // 2

        def make_kbody(h):
            def kbody(kk, accs):
                accs = list(accs)
                for dk in range(KU):
                    k = kk * KU + dk
                    for cb in range(HB):
                        c0 = h * 64 + cb * 16
                        w = wt_v[k, pl.ds(c0, 16)]
                        for n in range(BATCH):
                            r = rows_v[buf, n * K + k, pl.ds(c0, 16)]
                            accs[n * HB + cb] = accs[n * HB + cb] + r * w
                return tuple(accs)
            return kbody

        for h in range(2):
            accs = lax.fori_loop(
                0, K // KU, make_kbody(h),
                tuple(jnp.zeros((16,), jnp.float32)
                      for _ in range(BATCH * HB)))
            for n in range(BATCH):
                row = off * BATCH + n
                for cb in range(HB):
                    out_v[slot, row, pl.ds(h * 64 + cb * 16, 16)] = \
                        accs[n * HB + cb]

    fire_rows(0, 0)
    fire_flush(0, 0)
    fire_flush(1, OUTS)

    def body(it, carry):
        g0 = STEP * it

        @pl.when(g0 == 40)
        def _():
            pltpu.sync_copy(idx_hbm.at[pl.ds(abase + 48, 40), :],
                            idx_v.at[pl.ds(0, 40), :])

        for slot in range(2):
            wait_flush(slot)
            for j in range(OUTS):
                g = g0 + slot * OUTS + j
                rs = (slot * OUTS + j) & 1
                wait_rows(rs)
                fire_rows(g + 1, rs ^ 1)
                compute(g, rs, slot, j)
            fire_flush(slot, g0 + slot * OUTS)
        return carry

    lax.fori_loop(0, B // STEP, body, 0, unroll=False)

    wait_rows(0)
    wait_flush(0)
    wait_flush(1)


def kernel(x, edge_index, W):
    xt = x[0].T
    wt = W.T
    idx = edge_index[0].reshape(-1)
    idx = jnp.pad(idx, (0, IDX_ROWS * RPG - N * K))
    idx2d = idx.reshape(IDX_ROWS, RPG)
    out = _cluster_conv(xt, idx2d, wt)
    return out[None]


# speedup vs baseline: 1.5739x; 1.5739x over previous
import functools

import jax
import jax.numpy as jnp
from jax import lax
from jax.experimental import pallas as pl
from jax.experimental.pallas import tpu as pltpu
from jax.experimental.pallas import tpu_sc as plsc

N = 10000
C = 128
K = 32
NC = 2
NS = 16
NW = NC * NS
BATCH = 4
RPG = BATCH * K
CB = C // 16
KU = 4

B = 80
OUTS = 1
STEP = 2 * OUTS
NB_TOT = N // BATCH
IDX_ROWS = NB_TOT + 8

XSTRIPE = 632

assert B % STEP == 0

_mesh = plsc.VectorSubcoreMesh(core_axis_name="c", subcore_axis_name="s")


@functools.partial(
    pl.kernel,
    mesh=_mesh,
    out_type=jax.ShapeDtypeStruct((N, C), jnp.float32),
    scratch_types=[
        pltpu.VMEM_SHARED((N, C), jnp.float32),
        pltpu.VMEM((B + 8, RPG), jnp.int32),
        pltpu.VMEM((2, RPG, C), jnp.float32),
        pltpu.VMEM((2, OUTS * BATCH, C), jnp.float32),
        pltpu.VMEM((K, C), jnp.float32),
        pltpu.SemaphoreType.DMA,
        pltpu.SemaphoreType.DMA,
        pltpu.SemaphoreType.DMA,
        pltpu.SemaphoreType.DMA,
    ],
)
def _cluster_conv(xt_hbm, idx_hbm, wt_hbm, out_hbm,
                  xt_s, idx_v, rows_v, out_v, wt_v,
                  semr0, semr1, semf0, semf1):
    cid = lax.axis_index("c")
    sid = lax.axis_index("s")
    wid = sid * NC + cid
    base_b = jnp.minimum(wid * B, NB_TOT - B)
    abase = jnp.minimum(wid * B, (NB_TOT - B) // 8 * 8)
    goff = base_b - abase

    @pl.when(sid < NS - 1)
    def _():
        pltpu.sync_copy(xt_hbm.at[pl.ds(sid * XSTRIPE, XSTRIPE), :],
                        xt_s.at[pl.ds(sid * XSTRIPE, XSTRIPE), :])

    @pl.when(sid == NS - 1)
    def _():
        pltpu.sync_copy(
            xt_hbm.at[pl.ds((NS - 1) * XSTRIPE, N - (NS - 1) * XSTRIPE), :],
            xt_s.at[pl.ds((NS - 1) * XSTRIPE, N - (NS - 1) * XSTRIPE), :])

    pltpu.sync_copy(idx_hbm.at[pl.ds(abase, B + 8), :], idx_v)
    pltpu.sync_copy(wt_hbm, wt_v)
    plsc.subcore_barrier()

    semr = (semr0, semr1)
    semf = (semf0, semf1)

    def fire_rows(g, buf):
        pltpu.async_copy(xt_s.at[idx_v.at[goff + g]], rows_v.at[buf],
                         semr[buf])

    def wait_rows(buf):
        pltpu.make_async_copy(
            xt_s.at[idx_v.at[0]], rows_v.at[buf], semr[buf]).wait()

    def fire_flush(slot, g_first):
        pltpu.async_copy(
            out_v.at[slot],
            out_hbm.at[pl.ds((base_b + g_first) * BATCH, OUTS * BATCH), :],
            semf[slot])

    def wait_flush(slot):
        pltpu.make_async_copy(
            out_v.at[slot],
            out_hbm.at[pl.ds(base_b * BATCH, OUTS * BATCH), :],
            semf[slot]).wait()

    def compute(g, buf, slot, off):
        HB = CB // 2

        def make_kbody(h):
            def kbody(kk, accs):
                accs = list(accs)
                for dk in range(KU):
                    k = kk * KU + dk
                    for cb in range(HB):
                        c0 = h * 64 + cb * 16
                        w = wt_v[k, pl.ds(c0, 16)]
                        for n in range(BATCH):
                            r = rows_v[buf, n * K + k, pl.ds(c0, 16)]
                            accs[n * HB + cb] = accs[n * HB + cb] + r * w
                return tuple(accs)
            return kbody

        for h in range(2):
            accs = lax.fori_loop(
                0, K // KU, make_kbody(h),
                tuple(jnp.zeros((16,), jnp.float32)
                      for _ in range(BATCH * HB)))
            for n in range(BATCH):
                row = off * BATCH + n
                for cb in range(HB):
                    out_v[slot, row, pl.ds(h * 64 + cb * 16, 16)] = \
                        accs[n * HB + cb]

    fire_rows(0, 0)
    fire_flush(0, 0)
    fire_flush(1, OUTS)

    def body(it, carry):
        g0 = STEP * it
        for slot in range(2):
            wait_flush(slot)
            for j in range(OUTS):
                g = g0 + slot * OUTS + j
                rs = (slot * OUTS + j) & 1
                wait_rows(rs)
                fire_rows(g + 1, rs ^ 1)
                compute(g, rs, slot, j)
            fire_flush(slot, g0 + slot * OUTS)
        return carry

    lax.fori_loop(0, B // STEP, body, 0, unroll=False)

    wait_rows(0)
    wait_flush(0)
    wait_flush(1)


def kernel(x, edge_index, W):
    xt = x[0].T
    wt = W.T
    idx = edge_index[0].reshape(-1)
    idx = jnp.pad(idx, (0, IDX_ROWS * RPG - N * K))
    idx2d = idx.reshape(IDX_ROWS, RPG)
    out = _cluster_conv(xt, idx2d, wt)
    return out[None]
